# baseline (speedup 1.0000x reference)
import jax
import jax.numpy as jnp
from jax.experimental import pallas as pl

_INPUT_LENGTH = 128
_NUM_CODEBOOK = 4096
_BATCH = 4096

_Q_BLK = 256
_K_CHUNK = 2048


def _lvq_kernel(x_ref, c_ref, out_ref):
    xb = x_ref[...]
    cb = c_ref[...]
    x2 = jnp.sum(xb * xb, axis=1, keepdims=True)
    c2 = jnp.sum(cb * cb, axis=1)[None, :]
    dot = jax.lax.dot_general(
        xb, cb,
        dimension_numbers=(((1,), (1,)), ((), ())),
        preferred_element_type=jnp.float32,
    )
    d2 = (x2 + c2) - 2.0 * dot
    dist = jnp.sqrt(jnp.maximum(d2, 0.0))

    d_lo = dist[:, :_K_CHUNK]
    d_hi = dist[:, _K_CHUNK:]
    m_lo = jnp.min(d_lo, axis=1)
    i_lo = jnp.argmin(d_lo, axis=1).astype(jnp.int32)
    carry = m_lo.astype(jnp.bfloat16).astype(jnp.float32)
    m_hi = jnp.min(d_hi, axis=1)
    i_hi = jnp.argmin(d_hi, axis=1).astype(jnp.int32) + _K_CHUNK
    winner = jnp.where(m_hi < carry, i_hi, i_lo)
    out_ref[0, 0, :] = jnp.bitwise_and(winner, 1)


def kernel(x, codebook_vectors):
    n_blocks = _BATCH // _Q_BLK
    out = pl.pallas_call(
        _lvq_kernel,
        grid=(n_blocks,),
        in_specs=[
            pl.BlockSpec((_Q_BLK, _INPUT_LENGTH), lambda i: (i, 0)),
            pl.BlockSpec((_NUM_CODEBOOK, _INPUT_LENGTH), lambda i: (0, 0)),
        ],
        out_specs=pl.BlockSpec((1, 1, _Q_BLK), lambda i: (i, 0, 0)),
        out_shape=jax.ShapeDtypeStruct((n_blocks, 1, _Q_BLK), jnp.int32),
    )(x, codebook_vectors)
    return out.reshape(_BATCH)


# speedup vs baseline: 1.6247x; 1.6247x over previous
import jax
import jax.numpy as jnp
from jax.experimental import pallas as pl

_INPUT_LENGTH = 128
_NUM_CODEBOOK = 4096
_BATCH = 4096

_Q_BLK = 2048
_K_CHUNK = 2048
_K_TILE = 128
_K_STRIP = 512


def _lvq_kernel(x_ref, c_ref, out_ref):
    xb = x_ref[...]
    xb2 = xb + xb
    x2 = jnp.sum(xb * xb, axis=1, keepdims=True)
    lane = jax.lax.broadcasted_iota(jnp.int32, (_Q_BLK, _K_TILE), 1)
    big = jnp.float32(jnp.inf)

    def chunk_argmin(base):
        acc_v = jnp.full((_Q_BLK, _K_TILE), big, jnp.float32)
        acc_t = jnp.zeros((_Q_BLK, _K_TILE), jnp.int32)
        for w in range(_K_CHUNK // _K_STRIP):
            s = base + w * _K_STRIP
            cs = c_ref[pl.ds(s, _K_STRIP), :]
            dot2 = jax.lax.dot_general(
                xb2, cs,
                dimension_numbers=(((1,), (1,)), ((), ())),
                preferred_element_type=jnp.float32,
            )
            c2s = jnp.sum(cs * cs, axis=1)[None, :]
            d2 = (x2 + c2s) - dot2
            dist = jnp.sqrt(jnp.maximum(d2, 0.0))
            for u in range(_K_STRIP // _K_TILE):
                t = w * (_K_STRIP // _K_TILE) + u
                dtile = dist[:, u * _K_TILE:(u + 1) * _K_TILE]
                lt = dtile < acc_v
                acc_v = jnp.where(lt, dtile, acc_v)
                acc_t = jnp.where(lt, t, acc_t)
        m = jnp.min(acc_v, axis=1)
        idx = base + acc_t * _K_TILE + lane
        i = jnp.min(jnp.where(acc_v == m[:, None], idx, _NUM_CODEBOOK),
                    axis=1)
        return m, i

    m_lo, i_lo = chunk_argmin(0)
    carry = m_lo.astype(jnp.bfloat16).astype(jnp.float32)
    m_hi, i_hi = chunk_argmin(_K_CHUNK)
    winner = jnp.where(m_hi < carry, i_hi, i_lo)
    out_ref[0, 0, :] = jnp.bitwise_and(winner, 1)


def kernel(x, codebook_vectors):
    n_blocks = _BATCH // _Q_BLK
    out = pl.pallas_call(
        _lvq_kernel,
        grid=(n_blocks,),
        in_specs=[
            pl.BlockSpec((_Q_BLK, _INPUT_LENGTH), lambda i: (i, 0)),
            pl.BlockSpec((_NUM_CODEBOOK, _INPUT_LENGTH), lambda i: (0, 0)),
        ],
        out_specs=pl.BlockSpec((1, 1, _Q_BLK), lambda i: (i, 0, 0)),
        out_shape=jax.ShapeDtypeStruct((n_blocks, 1, _Q_BLK), jnp.int32),
    )(x, codebook_vectors)
    return out.reshape(_BATCH)
